# Initial kernel scaffold; baseline (speedup 1.0000x reference)
import functools

import jax
import jax.numpy as jnp
from jax.experimental import pallas as pl
from jax.experimental.pallas import tpu as pltpu

NF = 384


def _gather_body(idx_ref, x_ref, o_ref, *, C):
    i = pl.program_id(0)
    v = idx_ref[i]

    @pl.when(v < C)
    def _copy():
        o_ref[...] = x_ref[...]

    @pl.when(v >= C)
    def _zero():
        o_ref[...] = jnp.zeros_like(o_ref)


def kernel(x, indices):
    B, C, H, W = x.shape
    HW = H * W
    x3 = x.reshape(B, C, HW)

    grid_spec = pltpu.PrefetchScalarGridSpec(
        num_scalar_prefetch=1,
        grid=(NF,),
        in_specs=[
            pl.BlockSpec(
                (B, 1, HW),
                lambda i, idx_ref: (0, jnp.minimum(idx_ref[i], C - 1), 0),
            )
        ],
        out_specs=pl.BlockSpec((B, 1, HW), lambda i, idx_ref: (0, i, 0)),
    )
    out = pl.pallas_call(
        functools.partial(_gather_body, C=C),
        grid_spec=grid_spec,
        out_shape=jax.ShapeDtypeStruct((B, NF, HW), x.dtype),
    )(indices, x3)
    return out.reshape(B, NF, H, W)


# speedup vs baseline: 1.0102x; 1.0102x over previous
import functools

import jax
import jax.numpy as jnp
from jax.experimental import pallas as pl
from jax.experimental.pallas import tpu as pltpu

NF = 384


def _gather_body(idx_ref, x_ref, o_ref, *, C):
    i = pl.program_id(0)
    v = idx_ref[i]

    @pl.when(v < C)
    def _copy():
        o_ref[...] = x_ref[...]

    @pl.when(v >= C)
    def _zero():
        o_ref[...] = jnp.zeros_like(o_ref)


def kernel(x, indices):
    B, C, H, W = x.shape
    HW = H * W
    x4 = x.reshape(B, C, 1, HW)

    grid_spec = pltpu.PrefetchScalarGridSpec(
        num_scalar_prefetch=1,
        grid=(NF,),
        in_specs=[
            pl.BlockSpec(
                (B, 1, 1, HW),
                lambda i, idx_ref: (0, jnp.minimum(idx_ref[i], C - 1), 0, 0),
            )
        ],
        out_specs=pl.BlockSpec((B, 1, 1, HW), lambda i, idx_ref: (0, i, 0, 0)),
    )
    out = pl.pallas_call(
        functools.partial(_gather_body, C=C),
        grid_spec=grid_spec,
        out_shape=jax.ShapeDtypeStruct((B, NF, 1, HW), x.dtype),
    )(indices, x4)
    return out.reshape(B, NF, H, W)


# speedup vs baseline: 1.1324x; 1.1210x over previous
import functools

import jax
import jax.numpy as jnp
from jax.experimental import pallas as pl
from jax.experimental.pallas import tpu as pltpu

NF = 384


def _gather_body(idx_ref, x_ref, o_ref, *, C):
    i = pl.program_id(0)
    v = idx_ref[i]

    @pl.when(v < C)
    def _copy():
        o_ref[...] = x_ref[...]

    @pl.when(v >= C)
    def _zero():
        o_ref[...] = jnp.zeros_like(o_ref)


def kernel(x, indices):
    B, C, H, W = x.shape

    grid_spec = pltpu.PrefetchScalarGridSpec(
        num_scalar_prefetch=1,
        grid=(NF,),
        in_specs=[
            pl.BlockSpec(
                (B, 1, H, W),
                lambda i, idx_ref: (0, jnp.minimum(idx_ref[i], C - 1), 0, 0),
            )
        ],
        out_specs=pl.BlockSpec((B, 1, H, W), lambda i, idx_ref: (0, i, 0, 0)),
    )
    return pl.pallas_call(
        functools.partial(_gather_body, C=C),
        grid_spec=grid_spec,
        out_shape=jax.ShapeDtypeStruct((B, NF, H, W), x.dtype),
    )(indices, x)


# speedup vs baseline: 1.6719x; 1.4763x over previous
import functools

import jax
import jax.numpy as jnp
from jax.experimental import pallas as pl
from jax.experimental.pallas import tpu as pltpu

NF = 384
K = 8


def _gather_body(idx_ref, *refs, C):
    x_refs = refs[:K]
    o_ref = refs[K]
    i = pl.program_id(0)
    for j in range(K):
        v = idx_ref[i * K + j]

        @pl.when(v < C)
        def _copy(j=j):
            o_ref[:, j, :, :] = x_refs[j][:, 0, :, :]

        @pl.when(v >= C)
        def _zero(j=j):
            o_ref[:, j, :, :] = jnp.zeros_like(o_ref[:, j, :, :])


def kernel(x, indices):
    B, C, H, W = x.shape

    def make_in_spec(j):
        return pl.BlockSpec(
            (B, 1, H, W),
            lambda i, idx_ref: (0, jnp.minimum(idx_ref[i * K + j], C - 1), 0, 0),
        )

    grid_spec = pltpu.PrefetchScalarGridSpec(
        num_scalar_prefetch=1,
        grid=(NF // K,),
        in_specs=[make_in_spec(j) for j in range(K)],
        out_specs=pl.BlockSpec((B, K, H, W), lambda i, idx_ref: (0, i, 0, 0)),
    )
    return pl.pallas_call(
        functools.partial(_gather_body, C=C),
        grid_spec=grid_spec,
        out_shape=jax.ShapeDtypeStruct((B, NF, H, W), x.dtype),
    )(indices, *([x] * K))
